# baseline (speedup 1.0000x reference)
import functools

import jax
import jax.numpy as jnp
from jax import lax
from jax.experimental import pallas as pl
from jax.experimental.pallas import tpu as pltpu
from jax.experimental.pallas import tpu_sc as plsc

B = 16384
D = 64
L = 16
NC = 2
NS = 16
NW = NC * NS
BPW = B // NW
CHUNK = 128
NCHUNK = BPW // CHUNK
GPC = CHUNK // L


def _make_sc_kernel():
    mesh = plsc.VectorSubcoreMesh(core_axis_name="c", subcore_axis_name="s")

    @functools.partial(
        pl.kernel,
        mesh=mesh,
        out_type=jax.ShapeDtypeStruct((B,), jnp.float32),
        compiler_params=pltpu.CompilerParams(needs_layout_passes=False,
                                             use_tc_tiling_on_sc=False),
        scratch_types=[
            pltpu.VMEM((BPW, 2), jnp.int32),
            pltpu.VMEM((NCHUNK, CHUNK), jnp.int32),
            pltpu.VMEM((NCHUNK, CHUNK), jnp.int32),
            pltpu.VMEM((BPW, D), jnp.float32),
            pltpu.VMEM((BPW, D), jnp.float32),
            pltpu.VMEM((BPW,), jnp.float32),
            pltpu.SemaphoreType.DMA,
            pltpu.SemaphoreType.DMA,
            pltpu.SemaphoreType.DMA,
            pltpu.SemaphoreType.DMA,
        ],
    )
    def sc_body(ut_hbm, it_hbm, inp_hbm, out_hbm,
                raw_v, uix_v, iix_v, ur_v, ir_v, out_v, *sems):
        wid = lax.axis_index("s") * NC + lax.axis_index("c")
        pltpu.sync_copy(inp_hbm.at[wid], raw_v)

        lane = lax.iota(jnp.int32, L)
        col_u = jnp.zeros((L,), jnp.int32)
        col_i = jnp.ones((L,), jnp.int32)
        copies = []
        for j in range(NCHUNK):
            for g in range(GPC):
                rows = jnp.full((L,), j * CHUNK + g * L, jnp.int32) + lane
                uix_v[j, pl.ds(g * L, L)] = plsc.load_gather(raw_v, [rows, col_u])
                iix_v[j, pl.ds(g * L, L)] = plsc.load_gather(raw_v, [rows, col_i])
            copies.append((
                pltpu.async_copy(ut_hbm.at[uix_v.at[j]],
                                 ur_v.at[pl.ds(j * CHUNK, CHUNK)], sems[j]),
                pltpu.async_copy(it_hbm.at[iix_v.at[j]],
                                 ir_v.at[pl.ds(j * CHUNK, CHUNK)], sems[j]),
            ))

        def group(g, carry):
            row0 = pl.multiple_of(g * L, L)
            rows = row0 + lane
            acc = jnp.zeros((L,), jnp.float32)
            for col in range(D):
                cc = jnp.full((L,), col, jnp.int32)
                acc = acc + (plsc.load_gather(ur_v, [rows, cc])
                             * plsc.load_gather(ir_v, [rows, cc]))
            out_v[pl.ds(row0, L)] = acc
            return carry

        for j in range(NCHUNK):
            cu, ci = copies[j]
            cu.wait()
            ci.wait()
            lax.fori_loop(j * GPC, (j + 1) * GPC, group, 0)

        base = pl.multiple_of(wid * BPW, BPW)
        pltpu.sync_copy(out_v, out_hbm.at[pl.ds(base, BPW)])

    return sc_body


_sc_kernel = _make_sc_kernel()


def kernel(inputs, user_embedding, item_embedding):
    inp3 = inputs.reshape(NW, BPW, 2)
    return _sc_kernel(user_embedding, item_embedding, inp3)


# speedup vs baseline: 2.0277x; 2.0277x over previous
import functools

import jax
import jax.numpy as jnp
from jax import lax
from jax.experimental import pallas as pl
from jax.experimental.pallas import tpu as pltpu
from jax.experimental.pallas import tpu_sc as plsc

B = 16384
D = 64
V = 100000
L = 16
NC = 2
NS = 16
NW = NC * NS
CPW = D // NW
CHUNK = 2048
NCHUNK = B // CHUNK
PAIRS = CHUNK // (2 * L)


def _make_sc_kernel():
    mesh = plsc.VectorSubcoreMesh(core_axis_name="c", subcore_axis_name="s")

    @functools.partial(
        pl.kernel,
        mesh=mesh,
        out_type=jax.ShapeDtypeStruct((NW, B), jnp.float32),
        compiler_params=pltpu.CompilerParams(needs_layout_passes=False,
                                             use_tc_tiling_on_sc=True),
        scratch_types=[
            pltpu.VMEM((V,), jnp.float32),
            pltpu.VMEM((B // 2,), jnp.int32),
            pltpu.VMEM((B,), jnp.float32),
            pltpu.VMEM((CHUNK,), jnp.int32),
            pltpu.SemaphoreType.DMA,
        ],
    )
    def sc_body(ut_hbm, it_hbm, uix_hbm, iix_hbm, out_hbm,
                col_v, uvb_v, acc_v, ixc_v, sem):
        wid = lax.axis_index("s") * NC + lax.axis_index("c")

        for r in range(CPW):
            c = wid * CPW + r

            pltpu.async_copy(ut_hbm.at[c], col_v, sem).wait()

            def uchunk(k, carry):
                base = pl.multiple_of(k * CHUNK, CHUNK)
                pltpu.sync_copy(uix_hbm.at[pl.ds(base, CHUNK)], ixc_v)

                def upair(g, carry2):
                    off = pl.multiple_of(g * 2 * L, 2 * L)
                    ix0 = ixc_v[pl.ds(off, L)]
                    ix1 = ixc_v[pl.ds(off + L, L)]
                    v0 = plsc.load_gather(col_v, [ix0])
                    v1 = plsc.load_gather(col_v, [ix1])
                    packed = plsc.pack(v0, v1,
                                       format=plsc.PackFormat.INTERLEAVED)
                    uvb_v[pl.ds((base + off) // 2, L)] = plsc.bitcast(
                        packed, jnp.int32)
                    return carry2

                lax.fori_loop(0, PAIRS, upair, 0)
                return carry

            lax.fori_loop(0, NCHUNK, uchunk, 0)

            pltpu.async_copy(it_hbm.at[c], col_v, sem).wait()

            def ichunk(k, carry):
                base = pl.multiple_of(k * CHUNK, CHUNK)
                pltpu.sync_copy(iix_hbm.at[pl.ds(base, CHUNK)], ixc_v)

                def ipair(g, carry2):
                    off = pl.multiple_of(g * 2 * L, 2 * L)
                    pos = base + off
                    ix0 = ixc_v[pl.ds(off, L)]
                    ix1 = ixc_v[pl.ds(off + L, L)]
                    iv0 = plsc.load_gather(col_v, [ix0])
                    iv1 = plsc.load_gather(col_v, [ix1])
                    uu = plsc.bitcast(uvb_v[pl.ds(pos // 2, L)], jnp.bfloat16)
                    u0, u1 = plsc.unpack(uu,
                                         format=plsc.PackFormat.INTERLEAVED)
                    p0 = u0.astype(jnp.float32) * iv0
                    p1 = u1.astype(jnp.float32) * iv1
                    if r == 0:
                        acc_v[pl.ds(pos, L)] = p0
                        acc_v[pl.ds(pos + L, L)] = p1
                    else:
                        acc_v[pl.ds(pos, L)] = acc_v[pl.ds(pos, L)] + p0
                        acc_v[pl.ds(pos + L, L)] = acc_v[pl.ds(pos + L, L)] + p1
                    return carry2

                lax.fori_loop(0, PAIRS, ipair, 0)
                return carry

            lax.fori_loop(0, NCHUNK, ichunk, 0)

        pltpu.sync_copy(acc_v, out_hbm.at[wid])

    return sc_body


_sc_kernel = _make_sc_kernel()


def kernel(inputs, user_embedding, item_embedding):
    prods = _sc_kernel(user_embedding.T, item_embedding.T,
                       inputs[:, 0], inputs[:, 1])
    return jnp.sum(prods, axis=0)


# speedup vs baseline: 2.9983x; 1.4787x over previous
import functools

import jax
import jax.numpy as jnp
from jax import lax
from jax.experimental import pallas as pl
from jax.experimental.pallas import tpu as pltpu
from jax.experimental.pallas import tpu_sc as plsc

B = 16384
D = 64
V = 100000
L = 16
NC = 2
NS = 16
NW = NC * NS
CPW = D // NW
CHUNK = 4096
NCH = B // CHUNK
GR = CHUNK // L


def _make_sc_kernel():
    mesh = plsc.VectorSubcoreMesh(core_axis_name="c", subcore_axis_name="s")

    @functools.partial(
        pl.kernel,
        mesh=mesh,
        out_type=jax.ShapeDtypeStruct((D, B), jnp.float32),
        compiler_params=pltpu.CompilerParams(needs_layout_passes=False,
                                             use_tc_tiling_on_sc=True),
        scratch_types=[
            pltpu.VMEM((V,), jnp.float32),
            pltpu.VMEM((B,), jnp.float32),
            pltpu.VMEM((2, CHUNK), jnp.int32),
            pltpu.SemaphoreType.DMA,
            pltpu.SemaphoreType.DMA,
            pltpu.SemaphoreType.DMA,
            pltpu.SemaphoreType.DMA,
        ],
    )
    def sc_body(ut_hbm, it_hbm, uix_hbm, iix_hbm, out_hbm,
                col_v, val_v, ixc_v, semc, semi0, semi1, semo):
        wid = lax.axis_index("s") * NC + lax.axis_index("c")
        semi = (semi0, semi1)
        out_cp = None

        for r in range(CPW):
            c = wid * CPW + r

            for tbl in range(2):
                table = ut_hbm if tbl == 0 else it_hbm
                ix_hbm = uix_hbm if tbl == 0 else iix_hbm
                ccp = pltpu.async_copy(table.at[c], col_v, semc)
                cps = {0: pltpu.async_copy(ix_hbm.at[pl.ds(0, CHUNK)],
                                           ixc_v.at[0], semi[0])}
                ccp.wait()
                if tbl == 0 and out_cp is not None:
                    out_cp.wait()
                for k in range(NCH):
                    if k + 1 < NCH:
                        nb = (k + 1) % 2
                        cps[k + 1] = pltpu.async_copy(
                            ix_hbm.at[pl.ds((k + 1) * CHUNK, CHUNK)],
                            ixc_v.at[nb], semi[nb])
                    cps[k].wait()
                    base = k * CHUNK
                    buf = k % 2

                    if tbl == 0:
                        @plsc.parallel_loop(0, GR, unroll=8)
                        def ubody(g, base=base, buf=buf):
                            off = pl.multiple_of(g * L, L)
                            ix = ixc_v[buf, pl.ds(off, L)]
                            val_v[pl.ds(base + off, L)] = (
                                plsc.load_gather(col_v, [ix]))
                    else:
                        @plsc.parallel_loop(0, GR, unroll=8)
                        def ibody(g, base=base, buf=buf):
                            off = pl.multiple_of(g * L, L)
                            pos = base + off
                            ix = ixc_v[buf, pl.ds(off, L)]
                            iv = plsc.load_gather(col_v, [ix])
                            val_v[pl.ds(pos, L)] = val_v[pl.ds(pos, L)] * iv

            out_cp = pltpu.async_copy(val_v, out_hbm.at[c], semo)

        out_cp.wait()

    return sc_body


_sc_kernel = _make_sc_kernel()


def kernel(inputs, user_embedding, item_embedding):
    prods = _sc_kernel(user_embedding.T, item_embedding.T,
                       inputs[:, 0], inputs[:, 1])
    return jnp.sum(prods, axis=0)
